# baseline (speedup 1.0000x reference)
import jax
import jax.numpy as jnp
from jax.experimental import pallas as pl
from jax.experimental.pallas import tpu as pltpu

B, N, D, R = 4, 2048, 256, 3
TN = 512
TC = 256


def _score_body(nodes_ref, w_ref, b_ref, adj_ref, out_ref):
    r = pl.program_id(2)
    x = nodes_ref[0]
    wr = w_ref[0]
    xw = jnp.dot(x, wr, preferred_element_type=jnp.float32)
    a = adj_ref[0, 0]
    part = jnp.dot(a, xw, preferred_element_type=jnp.float32)
    part = part.reshape(1, 1, TN)

    @pl.when(r == 0)
    def _():
        out_ref[...] = part

    @pl.when(r > 0)
    def _():
        out_ref[...] += part

    @pl.when(r == R - 1)
    def _():
        out_ref[...] = jnp.tanh(out_ref[...] + b_ref[0])


def _rank_body(score_full_ref, score_col_ref, nums_ref, rank_ref, mask_ref, k_ref):
    ct = pl.program_id(1)
    s_full = score_full_ref[0]
    s_col = score_col_ref[0]
    st = s_full.reshape(N, 1)
    row = jax.lax.broadcasted_iota(jnp.int32, (N, 1), 0)
    col = jax.lax.broadcasted_iota(jnp.int32, (1, TC), 1) + ct * TC
    gt = st > s_col
    tie = (st == s_col) & (row < col)
    cmp = (gt | tie).astype(jnp.float32)
    ones = jnp.ones((1, N), dtype=jnp.float32)
    rank = jnp.dot(ones, cmp, preferred_element_type=jnp.float32)
    rank = rank.astype(jnp.int32)
    rank_ref[...] = rank.reshape(1, 1, TC)
    num = nums_ref[pl.program_id(0)]
    k = jnp.ceil(0.5 * num.astype(jnp.float32)).astype(jnp.int32)
    mask_ref[...] = (rank < k).astype(jnp.float32).reshape(1, 1, TC)

    @pl.when(ct == 0)
    def _():
        k_ref[pl.program_id(0)] = k


def _gather_body(rank_ref, score_ref, nums_ref, idx_ref, ks_ref):
    pt = pl.program_id(1)
    rank = rank_ref[0]
    s = score_ref[0]
    num = nums_ref[pl.program_id(0)]
    k = jnp.ceil(0.5 * num.astype(jnp.float32)).astype(jnp.int32)
    rt = rank.reshape(N, 1)
    p = jax.lax.broadcasted_iota(jnp.int32, (1, TC), 1) + pt * TC
    onehot = (rt == p).astype(jnp.float32)
    ivals = jax.lax.broadcasted_iota(jnp.int32, (1, N), 1).astype(jnp.float32)
    sorted_i = jnp.dot(ivals, onehot, preferred_element_type=jnp.float32)
    sorted_s = jnp.dot(s, onehot, preferred_element_type=jnp.float32)
    keep = p < k
    idx_ref[...] = jnp.where(keep, sorted_i.astype(jnp.int32), -1).reshape(1, 1, TC)
    ks_ref[...] = jnp.where(keep, sorted_s, 0.0).reshape(1, 1, TC)


def _hidden_body(nodes_ref, score_ref, mask_ref, out_ref):
    w = (score_ref[0] * mask_ref[0]).reshape(TN, 1)
    out_ref[0] = nodes_ref[0] * w


@jax.jit
def kernel(nodes, adjacency, batch_node_nums, W, b):
    score = pl.pallas_call(
        _score_body,
        grid=(B, N // TN, R),
        in_specs=[
            pl.BlockSpec((1, N, D), lambda bb, nt, r: (bb, 0, 0)),
            pl.BlockSpec((1, D, 1), lambda bb, nt, r: (r, 0, 0)),
            pl.BlockSpec(memory_space=pltpu.SMEM),
            pl.BlockSpec((1, 1, TN, N), lambda bb, nt, r: (bb, r, nt, 0)),
        ],
        out_specs=pl.BlockSpec((1, 1, TN), lambda bb, nt, r: (bb, 0, nt)),
        out_shape=jax.ShapeDtypeStruct((B, 1, N), jnp.float32),
    )(nodes, W, b, adjacency)

    rank, maskf, knum = pl.pallas_call(
        _rank_body,
        grid=(B, N // TC),
        in_specs=[
            pl.BlockSpec((1, 1, N), lambda bb, ct: (bb, 0, 0)),
            pl.BlockSpec((1, 1, TC), lambda bb, ct: (bb, 0, ct)),
            pl.BlockSpec(memory_space=pltpu.SMEM),
        ],
        out_specs=[
            pl.BlockSpec((1, 1, TC), lambda bb, ct: (bb, 0, ct)),
            pl.BlockSpec((1, 1, TC), lambda bb, ct: (bb, 0, ct)),
            pl.BlockSpec(memory_space=pltpu.SMEM, block_shape=(B,),
                         index_map=lambda bb, ct: (0,)),
        ],
        out_shape=[
            jax.ShapeDtypeStruct((B, 1, N), jnp.int32),
            jax.ShapeDtypeStruct((B, 1, N), jnp.float32),
            jax.ShapeDtypeStruct((B,), jnp.int32),
        ],
    )(score, score, batch_node_nums)

    keep_idx, keep_score = pl.pallas_call(
        _gather_body,
        grid=(B, N // TC),
        in_specs=[
            pl.BlockSpec((1, 1, N), lambda bb, pt: (bb, 0, 0)),
            pl.BlockSpec((1, 1, N), lambda bb, pt: (bb, 0, 0)),
            pl.BlockSpec(memory_space=pltpu.SMEM),
        ],
        out_specs=[
            pl.BlockSpec((1, 1, TC), lambda bb, pt: (bb, 0, pt)),
            pl.BlockSpec((1, 1, TC), lambda bb, pt: (bb, 0, pt)),
        ],
        out_shape=[
            jax.ShapeDtypeStruct((B, 1, N), jnp.int32),
            jax.ShapeDtypeStruct((B, 1, N), jnp.float32),
        ],
    )(rank, score, batch_node_nums)

    hidden = pl.pallas_call(
        _hidden_body,
        grid=(B, N // TN),
        in_specs=[
            pl.BlockSpec((1, TN, D), lambda bb, nt: (bb, nt, 0)),
            pl.BlockSpec((1, 1, TN), lambda bb, nt: (bb, 0, nt)),
            pl.BlockSpec((1, 1, TN), lambda bb, nt: (bb, 0, nt)),
        ],
        out_specs=pl.BlockSpec((1, TN, D), lambda bb, nt: (bb, nt, 0)),
        out_shape=jax.ShapeDtypeStruct((B, N, D), jnp.float32),
    )(nodes, score, maskf)

    return (hidden, knum.reshape(B), keep_idx.reshape(B, N),
            keep_score.reshape(B, N))


# speedup vs baseline: 1.0163x; 1.0163x over previous
import jax
import jax.numpy as jnp
from jax.experimental import pallas as pl
from jax.experimental.pallas import tpu as pltpu

B, N, D, R = 4, 2048, 256, 3
TN = 512
TB = 512
TP = 512
KMAX = N // 2


def _xw_body(nodes_ref, w_ref, out_ref):
    x = nodes_ref[0]
    w = w_ref[0]
    out_ref[...] = jnp.dot(x, w, preferred_element_type=jnp.float32).reshape(1, 1, N, 1)


def _score_body(xw_ref, b_ref, adj_ref, out_ref):
    r = pl.program_id(2)
    xwr = xw_ref[0, 0]
    a = adj_ref[0, 0]
    part = jnp.dot(a, xwr, preferred_element_type=jnp.float32)

    @pl.when(r == 0)
    def _():
        out_ref[0] = part

    @pl.when(r > 0)
    def _():
        out_ref[0] += part

    @pl.when(r == R - 1)
    def _():
        out_ref[0] = jnp.tanh(out_ref[0] + b_ref[0])


def _rank_body(score_full_ref, score_col_ref, nums_ref, rank_ref, mask_ref, k_ref):
    bi = pl.program_id(0)
    ct = pl.program_id(1)
    st = score_full_ref[0]
    s_row = score_col_ref[0].reshape(1, TB)
    irow = jax.lax.broadcasted_iota(jnp.int32, (N, 1), 0)
    jcol = jax.lax.broadcasted_iota(jnp.int32, (1, TB), 1) + ct * TB
    cmp = (st > s_row) | ((st == s_row) & (irow < jcol))
    ones = jnp.ones((1, N), dtype=jnp.float32)
    rank = jnp.dot(ones, cmp.astype(jnp.float32),
                   preferred_element_type=jnp.float32)
    rank = rank.astype(jnp.int32)
    rank_ref[...] = rank.reshape(1, 1, TB)
    num = nums_ref[bi]
    k = jnp.ceil(0.5 * num.astype(jnp.float32)).astype(jnp.int32)
    mask_ref[...] = (rank < k).astype(jnp.float32).reshape(1, 1, TB)

    @pl.when(ct == 0)
    def _():
        k_ref[bi] = k


def _gather_body(rank_ref, score_ref, nums_ref, idx_ref, ks_ref):
    bi = pl.program_id(0)
    pt = pl.program_id(1)

    @pl.when(pt * TP < KMAX)
    def _():
        rank_col = rank_ref[0].reshape(N, 1)
        s_row = score_ref[0].reshape(1, N)
        num = nums_ref[bi]
        k = jnp.ceil(0.5 * num.astype(jnp.float32)).astype(jnp.int32)
        p = jax.lax.broadcasted_iota(jnp.int32, (1, TP), 1) + pt * TP
        onehot = (rank_col == p).astype(jnp.float32)
        ivals = jax.lax.broadcasted_iota(jnp.int32, (1, N), 1).astype(jnp.float32)
        sorted_i = jnp.dot(ivals, onehot, preferred_element_type=jnp.float32)
        sorted_s = jnp.dot(s_row, onehot, preferred_element_type=jnp.float32)
        keep = p < k
        idx_ref[...] = jnp.where(keep, sorted_i.astype(jnp.int32), -1).reshape(1, 1, TP)
        ks_ref[...] = jnp.where(keep, sorted_s, 0.0).reshape(1, 1, TP)

    @pl.when(pt * TP >= KMAX)
    def _():
        idx_ref[...] = jnp.full((1, 1, TP), -1, jnp.int32)
        ks_ref[...] = jnp.zeros((1, 1, TP), jnp.float32)


def _hidden_body(nodes_ref, score_ref, mask_ref, out_ref):
    w = score_ref[0] * mask_ref[0].reshape(TN, 1)
    out_ref[0] = nodes_ref[0] * w


@jax.jit
def kernel(nodes, adjacency, batch_node_nums, W, b):
    xw = pl.pallas_call(
        _xw_body,
        grid=(B, R),
        in_specs=[
            pl.BlockSpec((1, N, D), lambda bb, r: (bb, 0, 0)),
            pl.BlockSpec((1, D, 1), lambda bb, r: (r, 0, 0)),
        ],
        out_specs=pl.BlockSpec((1, 1, N, 1), lambda bb, r: (bb, r, 0, 0)),
        out_shape=jax.ShapeDtypeStruct((B, R, N, 1), jnp.float32),
    )(nodes, W)

    score = pl.pallas_call(
        _score_body,
        grid=(B, N // TN, R),
        in_specs=[
            pl.BlockSpec((1, 1, N, 1), lambda bb, nt, r: (bb, r, 0, 0)),
            pl.BlockSpec(memory_space=pltpu.SMEM),
            pl.BlockSpec((1, 1, TN, N), lambda bb, nt, r: (bb, r, nt, 0)),
        ],
        out_specs=pl.BlockSpec((1, TN, 1), lambda bb, nt, r: (bb, nt, 0)),
        out_shape=jax.ShapeDtypeStruct((B, N, 1), jnp.float32),
    )(xw, b, adjacency)

    rank, maskf, knum = pl.pallas_call(
        _rank_body,
        grid=(B, N // TB),
        in_specs=[
            pl.BlockSpec((1, N, 1), lambda bb, ct: (bb, 0, 0)),
            pl.BlockSpec((1, TB, 1), lambda bb, ct: (bb, ct, 0)),
            pl.BlockSpec(memory_space=pltpu.SMEM),
        ],
        out_specs=[
            pl.BlockSpec((1, 1, TB), lambda bb, ct: (bb, 0, ct)),
            pl.BlockSpec((1, 1, TB), lambda bb, ct: (bb, 0, ct)),
            pl.BlockSpec(memory_space=pltpu.SMEM, block_shape=(B,),
                         index_map=lambda bb, ct: (0,)),
        ],
        out_shape=[
            jax.ShapeDtypeStruct((B, 1, N), jnp.int32),
            jax.ShapeDtypeStruct((B, 1, N), jnp.float32),
            jax.ShapeDtypeStruct((B,), jnp.int32),
        ],
    )(score, score, batch_node_nums)

    keep_idx, keep_score = pl.pallas_call(
        _gather_body,
        grid=(B, N // TP),
        in_specs=[
            pl.BlockSpec((1, 1, N), lambda bb, pt: (bb, 0, 0)),
            pl.BlockSpec((1, N, 1), lambda bb, pt: (bb, 0, 0)),
            pl.BlockSpec(memory_space=pltpu.SMEM),
        ],
        out_specs=[
            pl.BlockSpec((1, 1, TP), lambda bb, pt: (bb, 0, pt)),
            pl.BlockSpec((1, 1, TP), lambda bb, pt: (bb, 0, pt)),
        ],
        out_shape=[
            jax.ShapeDtypeStruct((B, 1, N), jnp.int32),
            jax.ShapeDtypeStruct((B, 1, N), jnp.float32),
        ],
    )(rank, score, batch_node_nums)

    hidden = pl.pallas_call(
        _hidden_body,
        grid=(B, N // TN),
        in_specs=[
            pl.BlockSpec((1, TN, D), lambda bb, nt: (bb, nt, 0)),
            pl.BlockSpec((1, TN, 1), lambda bb, nt: (bb, nt, 0)),
            pl.BlockSpec((1, 1, TN), lambda bb, nt: (bb, 0, nt)),
        ],
        out_specs=pl.BlockSpec((1, TN, D), lambda bb, nt: (bb, nt, 0)),
        out_shape=jax.ShapeDtypeStruct((B, N, D), jnp.float32),
    )(nodes, score, maskf)

    return (hidden, knum, keep_idx.reshape(B, N), keep_score.reshape(B, N))


# speedup vs baseline: 1.0473x; 1.0305x over previous
import jax
import jax.numpy as jnp
from jax.experimental import pallas as pl
from jax.experimental.pallas import tpu as pltpu

B, N, D, R = 4, 2048, 256, 3
TN = 512
TB = 512
TP = 512
KMAX = N // 2


def _xw_body(nodes_ref, w_ref, out_ref):
    x = nodes_ref[0]
    w = w_ref[0]
    out_ref[...] = jnp.dot(x, w, preferred_element_type=jnp.float32).reshape(1, 1, N, 1)


def _score_body(xw_ref, b_ref, adj_ref, out_ref):
    r = pl.program_id(2)
    xwr = xw_ref[0, 0]
    a = adj_ref[0, 0]
    part = jnp.sum(a, axis=1, keepdims=True) * xwr[0, 0]

    @pl.when(r == 0)
    def _():
        out_ref[0] = part

    @pl.when(r > 0)
    def _():
        out_ref[0] += part

    @pl.when(r == R - 1)
    def _():
        out_ref[0] = jnp.tanh(out_ref[0] + b_ref[0])


def _rank_body(score_full_ref, score_col_ref, nums_ref, rank_ref, mask_ref, k_ref):
    bi = pl.program_id(0)
    ct = pl.program_id(1)
    st = score_full_ref[0]
    s_row = score_col_ref[0].reshape(1, TB)
    irow = jax.lax.broadcasted_iota(jnp.int32, (N, 1), 0)
    jcol = jax.lax.broadcasted_iota(jnp.int32, (1, TB), 1) + ct * TB
    cmp = (st > s_row) | ((st == s_row) & (irow < jcol))
    ones = jnp.ones((1, N), dtype=jnp.float32)
    rank = jnp.dot(ones, cmp.astype(jnp.float32),
                   preferred_element_type=jnp.float32)
    rank = rank.astype(jnp.int32)
    rank_ref[...] = rank.reshape(1, 1, TB)
    num = nums_ref[bi]
    k = jnp.ceil(0.5 * num.astype(jnp.float32)).astype(jnp.int32)
    mask_ref[...] = (rank < k).astype(jnp.float32).reshape(1, 1, TB)

    @pl.when(ct == 0)
    def _():
        k_ref[bi] = k


def _gather_body(rank_ref, score_ref, nums_ref, idx_ref, ks_ref):
    bi = pl.program_id(0)
    pt = pl.program_id(1)

    @pl.when(pt * TP < KMAX)
    def _():
        rank_col = rank_ref[0].reshape(N, 1)
        s_row = score_ref[0].reshape(1, N)
        num = nums_ref[bi]
        k = jnp.ceil(0.5 * num.astype(jnp.float32)).astype(jnp.int32)
        p = jax.lax.broadcasted_iota(jnp.int32, (1, TP), 1) + pt * TP
        onehot = (rank_col == p).astype(jnp.float32)
        ivals = jax.lax.broadcasted_iota(jnp.int32, (1, N), 1).astype(jnp.float32)
        sorted_i = jnp.dot(ivals, onehot, preferred_element_type=jnp.float32)
        sorted_s = jnp.dot(s_row, onehot, preferred_element_type=jnp.float32)
        keep = p < k
        idx_ref[...] = jnp.where(keep, sorted_i.astype(jnp.int32), -1).reshape(1, 1, TP)
        ks_ref[...] = jnp.where(keep, sorted_s, 0.0).reshape(1, 1, TP)

    @pl.when(pt * TP >= KMAX)
    def _():
        idx_ref[...] = jnp.full((1, 1, TP), -1, jnp.int32)
        ks_ref[...] = jnp.zeros((1, 1, TP), jnp.float32)


def _hidden_body(nodes_ref, score_ref, mask_ref, out_ref):
    w = score_ref[0] * mask_ref[0].reshape(TN, 1)
    out_ref[0] = nodes_ref[0] * w


@jax.jit
def kernel(nodes, adjacency, batch_node_nums, W, b):
    xw = pl.pallas_call(
        _xw_body,
        grid=(B, R),
        in_specs=[
            pl.BlockSpec((1, N, D), lambda bb, r: (bb, 0, 0)),
            pl.BlockSpec((1, D, 1), lambda bb, r: (r, 0, 0)),
        ],
        out_specs=pl.BlockSpec((1, 1, N, 1), lambda bb, r: (bb, r, 0, 0)),
        out_shape=jax.ShapeDtypeStruct((B, R, N, 1), jnp.float32),
    )(nodes, W)

    score = pl.pallas_call(
        _score_body,
        grid=(B, N // TN, R),
        in_specs=[
            pl.BlockSpec((1, 1, N, 1), lambda bb, nt, r: (bb, r, 0, 0)),
            pl.BlockSpec(memory_space=pltpu.SMEM),
            pl.BlockSpec((1, 1, TN, N), lambda bb, nt, r: (bb, r, nt, 0)),
        ],
        out_specs=pl.BlockSpec((1, TN, 1), lambda bb, nt, r: (bb, nt, 0)),
        out_shape=jax.ShapeDtypeStruct((B, N, 1), jnp.float32),
    )(xw, b, adjacency)

    rank, maskf, knum = pl.pallas_call(
        _rank_body,
        grid=(B, N // TB),
        in_specs=[
            pl.BlockSpec((1, N, 1), lambda bb, ct: (bb, 0, 0)),
            pl.BlockSpec((1, TB, 1), lambda bb, ct: (bb, ct, 0)),
            pl.BlockSpec(memory_space=pltpu.SMEM),
        ],
        out_specs=[
            pl.BlockSpec((1, 1, TB), lambda bb, ct: (bb, 0, ct)),
            pl.BlockSpec((1, 1, TB), lambda bb, ct: (bb, 0, ct)),
            pl.BlockSpec(memory_space=pltpu.SMEM, block_shape=(B,),
                         index_map=lambda bb, ct: (0,)),
        ],
        out_shape=[
            jax.ShapeDtypeStruct((B, 1, N), jnp.int32),
            jax.ShapeDtypeStruct((B, 1, N), jnp.float32),
            jax.ShapeDtypeStruct((B,), jnp.int32),
        ],
    )(score, score, batch_node_nums)

    keep_idx, keep_score = pl.pallas_call(
        _gather_body,
        grid=(B, N // TP),
        in_specs=[
            pl.BlockSpec((1, 1, N), lambda bb, pt: (bb, 0, 0)),
            pl.BlockSpec((1, N, 1), lambda bb, pt: (bb, 0, 0)),
            pl.BlockSpec(memory_space=pltpu.SMEM),
        ],
        out_specs=[
            pl.BlockSpec((1, 1, TP), lambda bb, pt: (bb, 0, pt)),
            pl.BlockSpec((1, 1, TP), lambda bb, pt: (bb, 0, pt)),
        ],
        out_shape=[
            jax.ShapeDtypeStruct((B, 1, N), jnp.int32),
            jax.ShapeDtypeStruct((B, 1, N), jnp.float32),
        ],
    )(rank, score, batch_node_nums)

    hidden = pl.pallas_call(
        _hidden_body,
        grid=(B, N // TN),
        in_specs=[
            pl.BlockSpec((1, TN, D), lambda bb, nt: (bb, nt, 0)),
            pl.BlockSpec((1, TN, 1), lambda bb, nt: (bb, nt, 0)),
            pl.BlockSpec((1, 1, TN), lambda bb, nt: (bb, 0, nt)),
        ],
        out_specs=pl.BlockSpec((1, TN, D), lambda bb, nt: (bb, nt, 0)),
        out_shape=jax.ShapeDtypeStruct((B, N, D), jnp.float32),
    )(nodes, score, maskf)

    return (hidden, knum, keep_idx.reshape(B, N), keep_score.reshape(B, N))


# speedup vs baseline: 1.0659x; 1.0177x over previous
import jax
import jax.numpy as jnp
from jax.experimental import pallas as pl
from jax.experimental.pallas import tpu as pltpu

B, N, D, R = 4, 2048, 256, 3
TN = 512
TB = 512
TP = 512
KMAX = N // 2


def _xw_body(nodes_ref, w_ref, out_ref):
    x = nodes_ref[0]
    w = w_ref[0]
    out_ref[...] = jnp.dot(x, w, preferred_element_type=jnp.float32).reshape(1, 1, N, 1)


def _score_body(xw_ref, b_ref, adj_ref, adj2_ref, out_ref):
    r = pl.program_id(2)
    xwr = xw_ref[0, 0]
    a = adj_ref[0, 0]
    a2 = adj2_ref[0, 0]
    part = (jnp.concatenate([jnp.sum(a, axis=1, keepdims=True),
                             jnp.sum(a2, axis=1, keepdims=True)], axis=0)
            * xwr[0, 0])

    @pl.when(r == 0)
    def _():
        out_ref[0] = part

    @pl.when(r > 0)
    def _():
        out_ref[0] += part

    @pl.when(r == R - 1)
    def _():
        out_ref[0] = jnp.tanh(out_ref[0] + b_ref[0])


def _rank_body(score_full_ref, score_col_ref, nums_ref, rank_ref, mask_ref, k_ref):
    bi = pl.program_id(0)
    ct = pl.program_id(1)
    st = score_full_ref[0]
    s_row = score_col_ref[0].reshape(1, TB)
    irow = jax.lax.broadcasted_iota(jnp.int32, (N, 1), 0)
    jcol = jax.lax.broadcasted_iota(jnp.int32, (1, TB), 1) + ct * TB
    cmp = (st > s_row) | ((st == s_row) & (irow < jcol))
    ones = jnp.ones((1, N), dtype=jnp.float32)
    rank = jnp.dot(ones, cmp.astype(jnp.float32),
                   preferred_element_type=jnp.float32)
    rank = rank.astype(jnp.int32)
    rank_ref[...] = rank.reshape(1, 1, TB)
    num = nums_ref[bi]
    k = jnp.ceil(0.5 * num.astype(jnp.float32)).astype(jnp.int32)
    mask_ref[...] = (rank < k).astype(jnp.float32).reshape(1, 1, TB)

    @pl.when(ct == 0)
    def _():
        k_ref[bi] = k


def _gather_body(rank_ref, score_ref, nums_ref, idx_ref, ks_ref):
    bi = pl.program_id(0)
    pt = pl.program_id(1)

    @pl.when(pt * TP < KMAX)
    def _():
        rank_col = rank_ref[0].reshape(N, 1)
        s_row = score_ref[0].reshape(1, N)
        num = nums_ref[bi]
        k = jnp.ceil(0.5 * num.astype(jnp.float32)).astype(jnp.int32)
        p = jax.lax.broadcasted_iota(jnp.int32, (1, TP), 1) + pt * TP
        onehot = (rank_col == p).astype(jnp.float32)
        ivals = jax.lax.broadcasted_iota(jnp.int32, (1, N), 1).astype(jnp.float32)
        sorted_i = jnp.dot(ivals, onehot, preferred_element_type=jnp.float32)
        sorted_s = jnp.dot(s_row, onehot, preferred_element_type=jnp.float32)
        keep = p < k
        idx_ref[...] = jnp.where(keep, sorted_i.astype(jnp.int32), -1).reshape(1, 1, TP)
        ks_ref[...] = jnp.where(keep, sorted_s, 0.0).reshape(1, 1, TP)

    @pl.when(pt * TP >= KMAX)
    def _():
        idx_ref[...] = jnp.full((1, 1, TP), -1, jnp.int32)
        ks_ref[...] = jnp.zeros((1, 1, TP), jnp.float32)


def _hidden_body(nodes_ref, score_ref, mask_ref, out_ref):
    w = score_ref[0] * mask_ref[0].reshape(TN, 1)
    out_ref[0] = nodes_ref[0] * w


@jax.jit
def kernel(nodes, adjacency, batch_node_nums, W, b):
    xw = pl.pallas_call(
        _xw_body,
        grid=(B, R),
        in_specs=[
            pl.BlockSpec((1, N, D), lambda bb, r: (bb, 0, 0)),
            pl.BlockSpec((1, D, 1), lambda bb, r: (r, 0, 0)),
        ],
        out_specs=pl.BlockSpec((1, 1, N, 1), lambda bb, r: (bb, r, 0, 0)),
        out_shape=jax.ShapeDtypeStruct((B, R, N, 1), jnp.float32),
    )(nodes, W)

    score = pl.pallas_call(
        _score_body,
        grid=(B, N // TN, R),
        in_specs=[
            pl.BlockSpec((1, 1, N, 1), lambda bb, nt, r: (bb, r, 0, 0)),
            pl.BlockSpec(memory_space=pltpu.SMEM),
            pl.BlockSpec((1, 1, TN // 2, N), lambda bb, nt, r: (bb, r, 2 * nt, 0)),
            pl.BlockSpec((1, 1, TN // 2, N), lambda bb, nt, r: (bb, r, 2 * nt + 1, 0)),
        ],
        out_specs=pl.BlockSpec((1, TN, 1), lambda bb, nt, r: (bb, nt, 0)),
        out_shape=jax.ShapeDtypeStruct((B, N, 1), jnp.float32),
    )(xw, b, adjacency, adjacency)

    rank, maskf, knum = pl.pallas_call(
        _rank_body,
        grid=(B, N // TB),
        in_specs=[
            pl.BlockSpec((1, N, 1), lambda bb, ct: (bb, 0, 0)),
            pl.BlockSpec((1, TB, 1), lambda bb, ct: (bb, ct, 0)),
            pl.BlockSpec(memory_space=pltpu.SMEM),
        ],
        out_specs=[
            pl.BlockSpec((1, 1, TB), lambda bb, ct: (bb, 0, ct)),
            pl.BlockSpec((1, 1, TB), lambda bb, ct: (bb, 0, ct)),
            pl.BlockSpec(memory_space=pltpu.SMEM, block_shape=(B,),
                         index_map=lambda bb, ct: (0,)),
        ],
        out_shape=[
            jax.ShapeDtypeStruct((B, 1, N), jnp.int32),
            jax.ShapeDtypeStruct((B, 1, N), jnp.float32),
            jax.ShapeDtypeStruct((B,), jnp.int32),
        ],
    )(score, score, batch_node_nums)

    keep_idx, keep_score = pl.pallas_call(
        _gather_body,
        grid=(B, N // TP),
        in_specs=[
            pl.BlockSpec((1, 1, N), lambda bb, pt: (bb, 0, 0)),
            pl.BlockSpec((1, N, 1), lambda bb, pt: (bb, 0, 0)),
            pl.BlockSpec(memory_space=pltpu.SMEM),
        ],
        out_specs=[
            pl.BlockSpec((1, 1, TP), lambda bb, pt: (bb, 0, pt)),
            pl.BlockSpec((1, 1, TP), lambda bb, pt: (bb, 0, pt)),
        ],
        out_shape=[
            jax.ShapeDtypeStruct((B, 1, N), jnp.int32),
            jax.ShapeDtypeStruct((B, 1, N), jnp.float32),
        ],
    )(rank, score, batch_node_nums)

    hidden = pl.pallas_call(
        _hidden_body,
        grid=(B, N // TN),
        in_specs=[
            pl.BlockSpec((1, TN, D), lambda bb, nt: (bb, nt, 0)),
            pl.BlockSpec((1, TN, 1), lambda bb, nt: (bb, nt, 0)),
            pl.BlockSpec((1, 1, TN), lambda bb, nt: (bb, 0, nt)),
        ],
        out_specs=pl.BlockSpec((1, TN, D), lambda bb, nt: (bb, nt, 0)),
        out_shape=jax.ShapeDtypeStruct((B, N, D), jnp.float32),
    )(nodes, score, maskf)

    return (hidden, knum, keep_idx.reshape(B, N), keep_score.reshape(B, N))


# speedup vs baseline: 1.6151x; 1.5153x over previous
import jax
import jax.numpy as jnp
from jax.experimental import pallas as pl
from jax.experimental.pallas import tpu as pltpu

B, N, D, R = 4, 2048, 256, 3
TB = 256
TP = 512
KMAX = N // 2


def _xw_body(nodes_ref, w_ref, out_ref):
    x = nodes_ref[0]
    w = w_ref[0]
    out_ref[...] = jnp.dot(x, w, preferred_element_type=jnp.float32).reshape(1, 1, N, 1)


def _mega_body(xw_ref, b_ref, nums_ref, adj_ref, nodes_ref,
               hid_ref, k_ref, idx_ref, ks_ref, acc_ref):
    bi = pl.program_id(0)
    r = pl.program_id(1)
    part = jnp.dot(adj_ref[0, 0], xw_ref[0, 0],
                   preferred_element_type=jnp.float32)

    @pl.when(r == 0)
    def _():
        acc_ref[...] = part

    @pl.when(r > 0)
    def _():
        acc_ref[...] += part

    @pl.when(r == R - 1)
    def _():
        s = jnp.tanh(acc_ref[...] + b_ref[0])
        num = nums_ref[bi]
        k = jnp.ceil(0.5 * num.astype(jnp.float32)).astype(jnp.int32)
        k_ref[bi] = k
        s_row = s.reshape(1, N)
        irow = jax.lax.broadcasted_iota(jnp.int32, (N, 1), 0)
        ones = jnp.ones((1, N), dtype=jnp.float32)

        rank_chunks = []
        for c in range(N // TB):
            sj = jax.lax.slice(s_row, (0, c * TB), (1, (c + 1) * TB))
            jcol = jax.lax.broadcasted_iota(jnp.int32, (1, TB), 1) + c * TB
            cmp = (s > sj) | ((s == sj) & (irow < jcol))
            rank_chunks.append(jnp.dot(ones, cmp.astype(jnp.float32),
                                       preferred_element_type=jnp.float32))
        rank_row = jnp.concatenate(rank_chunks, axis=1).astype(jnp.int32)

        mask_col = (rank_row < k).astype(jnp.float32).reshape(N, 1)
        hid_ref[0] = nodes_ref[0] * (s * mask_col)

        rank_col = rank_row.reshape(N, 1)
        ivals = jax.lax.broadcasted_iota(jnp.int32, (1, N), 1).astype(jnp.float32)
        for c in range(KMAX // TP):
            p = jax.lax.broadcasted_iota(jnp.int32, (1, TP), 1) + c * TP
            onehot = (rank_col == p).astype(jnp.float32)
            sorted_i = jnp.dot(ivals, onehot, preferred_element_type=jnp.float32)
            sorted_s = jnp.dot(s_row, onehot, preferred_element_type=jnp.float32)
            keep = p < k
            idx_ref[0, 0, c * TP:(c + 1) * TP] = jnp.where(
                keep, sorted_i.astype(jnp.int32), -1).reshape(TP)
            ks_ref[0, 0, c * TP:(c + 1) * TP] = jnp.where(
                keep, sorted_s, 0.0).reshape(TP)
        idx_ref[0, 0, KMAX:] = jnp.full((N - KMAX,), -1, jnp.int32)
        ks_ref[0, 0, KMAX:] = jnp.zeros((N - KMAX,), jnp.float32)


@jax.jit
def kernel(nodes, adjacency, batch_node_nums, W, b):
    xw = pl.pallas_call(
        _xw_body,
        grid=(B, R),
        in_specs=[
            pl.BlockSpec((1, N, D), lambda bb, r: (bb, 0, 0)),
            pl.BlockSpec((1, D, 1), lambda bb, r: (r, 0, 0)),
        ],
        out_specs=pl.BlockSpec((1, 1, N, 1), lambda bb, r: (bb, r, 0, 0)),
        out_shape=jax.ShapeDtypeStruct((B, R, N, 1), jnp.float32),
    )(nodes, W)

    hidden, knum, keep_idx, keep_score = pl.pallas_call(
        _mega_body,
        grid=(B, R),
        in_specs=[
            pl.BlockSpec((1, 1, N, 1), lambda bb, r: (bb, r, 0, 0)),
            pl.BlockSpec(memory_space=pltpu.SMEM),
            pl.BlockSpec(memory_space=pltpu.SMEM),
            pl.BlockSpec((1, 1, N, N), lambda bb, r: (bb, r, 0, 0)),
            pl.BlockSpec((1, N, D), lambda bb, r: (bb, 0, 0)),
        ],
        out_specs=[
            pl.BlockSpec((1, N, D), lambda bb, r: (bb, 0, 0)),
            pl.BlockSpec(memory_space=pltpu.SMEM, block_shape=(B,),
                         index_map=lambda bb, r: (0,)),
            pl.BlockSpec((1, 1, N), lambda bb, r: (bb, 0, 0)),
            pl.BlockSpec((1, 1, N), lambda bb, r: (bb, 0, 0)),
        ],
        out_shape=[
            jax.ShapeDtypeStruct((B, N, D), jnp.float32),
            jax.ShapeDtypeStruct((B,), jnp.int32),
            jax.ShapeDtypeStruct((B, 1, N), jnp.int32),
            jax.ShapeDtypeStruct((B, 1, N), jnp.float32),
        ],
        scratch_shapes=[pltpu.VMEM((N, 1), jnp.float32)],
    )(xw, b, batch_node_nums, adjacency, nodes)

    return (hidden, knum, keep_idx.reshape(B, N), keep_score.reshape(B, N))


# speedup vs baseline: 1.9297x; 1.1947x over previous
import jax
import jax.numpy as jnp
from jax.experimental import pallas as pl
from jax.experimental.pallas import tpu as pltpu

B, N, D, R = 4, 2048, 256, 3
TB = 256
TP = 512
KMAX = N // 2


def _mega_body(w_ref, b_ref, nums_ref, adj_hi_ref, adj_lo_ref, nodes_ref,
               hid_ref, k_ref, idx_ref, ks_ref, acc_ref):
    bi = pl.program_id(0)
    r = pl.program_id(1)
    xwr = jnp.dot(nodes_ref[0], w_ref[0],
                  preferred_element_type=jnp.float32)
    part = jnp.concatenate(
        [jnp.dot(adj_hi_ref[0, 0], xwr, preferred_element_type=jnp.float32),
         jnp.dot(adj_lo_ref[0, 0], xwr, preferred_element_type=jnp.float32)],
        axis=0)

    @pl.when(r == 0)
    def _():
        acc_ref[...] = part

    @pl.when(r > 0)
    def _():
        acc_ref[...] += part

    @pl.when(r == R - 1)
    def _():
        s = jnp.tanh(acc_ref[...] + b_ref[0])
        num = nums_ref[bi]
        k = jnp.ceil(0.5 * num.astype(jnp.float32)).astype(jnp.int32)
        k_ref[bi] = k
        s_row = s.reshape(1, N)
        irow = jax.lax.broadcasted_iota(jnp.int32, (N, 1), 0)
        ones = jnp.ones((1, N), dtype=jnp.float32)

        rank_chunks = []
        for c in range(N // TB):
            sj = jax.lax.slice(s_row, (0, c * TB), (1, (c + 1) * TB))
            jcol = jax.lax.broadcasted_iota(jnp.int32, (1, TB), 1) + c * TB
            cmp = (s > sj) | ((s == sj) & (irow < jcol))
            rank_chunks.append(jnp.dot(ones, cmp.astype(jnp.float32),
                                       preferred_element_type=jnp.float32))
        rank_row = jnp.concatenate(rank_chunks, axis=1).astype(jnp.int32)

        mask_col = (rank_row < k).astype(jnp.float32).reshape(N, 1)
        hid_ref[0] = nodes_ref[0] * (s * mask_col)

        rank_col = rank_row.reshape(N, 1)
        ivals = jax.lax.broadcasted_iota(jnp.int32, (1, N), 1).astype(jnp.float32)
        for c in range(KMAX // TP):
            p = jax.lax.broadcasted_iota(jnp.int32, (1, TP), 1) + c * TP
            onehot = (rank_col == p).astype(jnp.float32)
            sorted_i = jnp.dot(ivals, onehot, preferred_element_type=jnp.float32)
            sorted_s = jnp.dot(s_row, onehot, preferred_element_type=jnp.float32)
            keep = p < k
            idx_ref[0, 0, c * TP:(c + 1) * TP] = jnp.where(
                keep, sorted_i.astype(jnp.int32), -1).reshape(TP)
            ks_ref[0, 0, c * TP:(c + 1) * TP] = jnp.where(
                keep, sorted_s, 0.0).reshape(TP)
        idx_ref[0, 0, KMAX:] = jnp.full((N - KMAX,), -1, jnp.int32)
        ks_ref[0, 0, KMAX:] = jnp.zeros((N - KMAX,), jnp.float32)


@jax.jit
def kernel(nodes, adjacency, batch_node_nums, W, b):
    hidden, knum, keep_idx, keep_score = pl.pallas_call(
        _mega_body,
        grid=(B, R),
        in_specs=[
            pl.BlockSpec((1, D, 1), lambda bb, r: (r, 0, 0)),
            pl.BlockSpec(memory_space=pltpu.SMEM),
            pl.BlockSpec(memory_space=pltpu.SMEM),
            pl.BlockSpec((1, 1, N // 2, N), lambda bb, r: (bb, r, 0, 0)),
            pl.BlockSpec((1, 1, N // 2, N), lambda bb, r: (bb, r, 1, 0)),
            pl.BlockSpec((1, N, D), lambda bb, r: (bb, 0, 0)),
        ],
        out_specs=[
            pl.BlockSpec((1, N, D), lambda bb, r: (bb, 0, 0)),
            pl.BlockSpec(memory_space=pltpu.SMEM, block_shape=(B,),
                         index_map=lambda bb, r: (0,)),
            pl.BlockSpec((1, 1, N), lambda bb, r: (bb, 0, 0)),
            pl.BlockSpec((1, 1, N), lambda bb, r: (bb, 0, 0)),
        ],
        out_shape=[
            jax.ShapeDtypeStruct((B, N, D), jnp.float32),
            jax.ShapeDtypeStruct((B,), jnp.int32),
            jax.ShapeDtypeStruct((B, 1, N), jnp.int32),
            jax.ShapeDtypeStruct((B, 1, N), jnp.float32),
        ],
        scratch_shapes=[pltpu.VMEM((N, 1), jnp.float32)],
    )(W, b, batch_node_nums, adjacency, adjacency, nodes)

    return (hidden, knum, keep_idx.reshape(B, N), keep_score.reshape(B, N))
